# baseline (speedup 1.0000x reference)
import functools
import math

import jax
import jax.numpy as jnp
from jax import lax
from jax.experimental import pallas as pl
from jax.experimental.pallas import tpu as pltpu
from jax.experimental.pallas import tpu_sc as plsc

_NC = 2
_NS = 16
_NW = _NC * _NS
_L = 16


@functools.lru_cache(maxsize=None)
def _build(seq, batch, dim, vocab):
    rows = seq * batch
    rows_w = rows // _NW
    seq_w = seq // _NW
    ch_seq = 8
    ch_rows = ch_seq * batch
    n_ch = seq_w // ch_seq
    k_sl = dim // _L
    scale = jnp.float32(math.sqrt(dim))

    assert rows % _NW == 0 and seq % _NW == 0
    assert seq_w % ch_seq == 0 and dim % _L == 0

    mesh = plsc.VectorSubcoreMesh(core_axis_name="c", subcore_axis_name="s")

    @functools.partial(
        pl.kernel,
        out_type=jax.ShapeDtypeStruct((seq, batch, dim), jnp.float32),
        mesh=mesh,
        scratch_types=[
            pltpu.VMEM((rows_w,), jnp.int32),
            pltpu.VMEM((2, ch_seq, 1, dim), jnp.float32),
            pltpu.VMEM((2, ch_rows, dim), jnp.float32),
            pltpu.SemaphoreType.DMA,
            pltpu.SemaphoreType.DMA,
            pltpu.SemaphoreType.DMA,
            pltpu.SemaphoreType.DMA,
            pltpu.SemaphoreType.DMA,
            pltpu.SemaphoreType.DMA,
        ],
    )
    def emb_pe(x_hbm, tab_hbm, pe_hbm, out_hbm, idx_v, pbuf, gbuf,
               gsem0, gsem1, psem0, psem1, osem0, osem1):
        wid = lax.axis_index("s") * _NC + lax.axis_index("c")
        row0 = wid * rows_w
        seq0 = wid * seq_w

        pltpu.sync_copy(x_hbm.at[pl.ds(row0, rows_w)], idx_v)

        gsems = (gsem0, gsem1)
        psems = (psem0, psem1)
        osems = (osem0, osem1)

        def start_gather(c, slot):
            return pltpu.async_copy(
                tab_hbm.at[idx_v.at[pl.ds(c * ch_rows, ch_rows)]],
                gbuf.at[slot],
                gsems[slot],
            )

        def start_pe(c, slot):
            return pltpu.async_copy(
                pe_hbm.at[pl.ds(seq0 + c * ch_seq, ch_seq)],
                pbuf.at[slot],
                psems[slot],
            )

        def start_outs(c, slot):
            return [
                pltpu.async_copy(
                    gbuf.at[slot, pl.ds(si * batch, batch)],
                    out_hbm.at[seq0 + c * ch_seq + si],
                    osems[slot],
                )
                for si in range(ch_seq)
            ]

        def compute(c, slot):
            @plsc.parallel_loop(0, ch_seq * k_sl, step=1, unroll=4)
            def _(i):
                si = i // k_sl
                off = (i % k_sl) * _L
                pvec = pbuf[slot, si, 0, pl.ds(off, _L)]
                r0 = si * batch
                for b in range(batch):
                    gbuf[slot, r0 + b, pl.ds(off, _L)] = (
                        gbuf[slot, r0 + b, pl.ds(off, _L)] * scale + pvec
                    )

        pending_outs = [None, None]
        gathers = [start_gather(0, 0), None]
        pes = [start_pe(0, 0), None]
        for c in range(n_ch):
            slot = c & 1
            nxt = 1 - slot
            if c + 1 < n_ch:
                if pending_outs[nxt] is not None:
                    for o in pending_outs[nxt]:
                        o.wait()
                    pending_outs[nxt] = None
                gathers[nxt] = start_gather(c + 1, nxt)
                pes[nxt] = start_pe(c + 1, nxt)
            gathers[slot].wait()
            pes[slot].wait()
            compute(c, slot)
            pending_outs[slot] = start_outs(c, slot)
        for po in pending_outs:
            if po is not None:
                for o in po:
                    o.wait()

    return emb_pe


def kernel(x, emb_table, pe):
    seq, batch = x.shape
    vocab, dim = emb_table.shape
    xf = x.reshape(seq * batch)
    return _build(seq, batch, dim, vocab)(xf, emb_table, pe)
